# baseline (speedup 1.0000x reference)
import functools

import jax
import jax.numpy as jnp
from jax import lax
from jax.experimental import pallas as pl
from jax.experimental.pallas import tpu as pltpu
from jax.experimental.pallas import tpu_sc as plsc

F32 = jnp.float32
I32 = jnp.int32

NC, NS, L = 2, 16, 16
NW = NC * NS
NNODE = 10000
NEDGE = 320000
FD = 128
KSEL = 5000
NOUT = 1000
NPAD1 = 10240
NPAD2 = 5120
EPW = NEDGE // NW
CB = 80
NCH = EPW // CB

_MESH = plsc.VectorSubcoreMesh(
    core_axis_name="c", subcore_axis_name="s", num_cores=NC, num_subcores=NS)
_SC_PARAMS = pltpu.CompilerParams(needs_layout_passes=False)


def _ids():
    c = lax.axis_index("c")
    s = lax.axis_index("s")
    return c, s, c * NS + s


def _fill1d(ref, n, val, dtype):
    v = jnp.full((L,), val, dtype)
    def body(i, _):
        ref[pl.ds(i * L, L)] = v
        return 0
    lax.fori_loop(0, n // L, body, 0)


def _zero2d(ref, rows):
    z = jnp.zeros((L,), F32)
    def body(i, _):
        for j in range(FD // L):
            ref[i, pl.ds(j * L, L)] = z
        return 0
    lax.fori_loop(0, rows, body, 0)


def _scale_rows(rows_ref, sc_ref, nrows):
    def body(r, _):
        gvec = sc_ref[pl.ds(r * L, L)]
        for t in range(L):
            i = r * L + t
            g = gvec[t]
            for j in range(FD // L):
                sl = (i, pl.ds(j * L, L))
                rows_ref[sl] = rows_ref[sl] * g
        return 0
    lax.fori_loop(0, nrows // L, body, 0)


def _red_geom(npad):
    stripe = npad // NS
    if stripe % 128 != 0:
        stripe = 512
    return stripe, npad // stripe


def _reduce16_store(degpriv, spm16, rbuf, outb, out_hbm, c, s, npad):
    stripe, nact = _red_geom(npad)
    pltpu.sync_copy(degpriv, spm16.at[pl.ds(s * npad, npad)])
    plsc.subcore_barrier()

    @pl.when(s < nact)
    def _():
        for t in range(NS):
            pltpu.sync_copy(spm16.at[pl.ds(t * npad + stripe * s, stripe)],
                            rbuf.at[pl.ds(t * stripe, stripe)])
        def body(j, _):
            acc = rbuf[pl.ds(j * L, L)]
            for t in range(1, NS):
                acc = acc + rbuf[pl.ds(t * stripe + j * L, L)]
            outb[pl.ds(j * L, L)] = acc
            return 0
        lax.fori_loop(0, stripe // L, body, 0)
        pltpu.sync_copy(outb, out_hbm.at[c, pl.ds(stripe * s, stripe)])


def _read_nch(cnt_h, csm, wid):
    pltpu.sync_copy(cnt_h.at[wid], csm)
    return csm[pl.ds(0, L)][0]


def _deg_body(npad, dst_h, w_h, degp_h, dch, wch, degpriv, spm16, rbuf, outb):
    c, s, wid = _ids()
    _fill1d(degpriv, npad, 0.0, F32)
    ebase = wid * EPW
    def chunk(i, _):
        pltpu.sync_copy(dst_h.at[pl.ds(ebase + i * CB, CB)], dch)
        pltpu.sync_copy(w_h.at[pl.ds(ebase + i * CB, CB)], wch)
        for j in range(CB // L):
            dv = dch[pl.ds(j * L, L)]
            wv = wch[pl.ds(j * L, L)]
            plsc.addupdate_scatter(degpriv, [dv], wv)
        return 0
    lax.fori_loop(0, NCH, chunk, 0)
    _reduce16_store(degpriv, spm16, rbuf, outb, degp_h, c, s, npad)


def _deg_call(dst, w, npad):
    stripe, _ = _red_geom(npad)
    body = functools.partial(_deg_body, npad)
    kfn = pl.kernel(
        body,
        out_type=jax.ShapeDtypeStruct((NC, npad), F32),
        mesh=_MESH,
        compiler_params=_SC_PARAMS,
        scratch_types=[
            pltpu.VMEM((CB,), I32),
            pltpu.VMEM((CB,), F32),
            pltpu.VMEM((npad,), F32),
            pltpu.VMEM_SHARED((NS * npad,), F32),
            pltpu.VMEM((NS * stripe,), F32),
            pltpu.VMEM((stripe,), F32),
        ],
    )
    return kfn(dst, w)


def _rsqrt_guard(d):
    dm = jnp.maximum(d, 1e-12)
    bits = lax.bitcast_convert_type(dm, I32)
    y = lax.bitcast_convert_type(jnp.int32(0x5F3759DF) - (bits >> 1), F32)
    for _ in range(3):
        y = y * (1.5 - 0.5 * dm * y * y)
    return jnp.where(d > 0, y, 0.0)


def _norm_body(npad, src_h, dst_h, w_h, degp_h, cnt_h, norm_h,
               da, db, disb, sch, dch, wch, ob, csm):
    c, s, wid = _ids()
    if cnt_h is None:
        nch = NCH
    else:
        nch = _read_nch(cnt_h, csm, wid)
    pltpu.sync_copy(degp_h.at[0], da)
    pltpu.sync_copy(degp_h.at[1], db)
    def dbody(i, _):
        sl = pl.ds(i * L, L)
        disb[sl] = _rsqrt_guard(da[sl] + db[sl])
        return 0
    lax.fori_loop(0, npad // L, dbody, 0)
    ebase = wid * EPW
    def chunk(i, _):
        pltpu.sync_copy(src_h.at[pl.ds(ebase + i * CB, CB)], sch)
        pltpu.sync_copy(dst_h.at[pl.ds(ebase + i * CB, CB)], dch)
        pltpu.sync_copy(w_h.at[pl.ds(ebase + i * CB, CB)], wch)
        for j in range(CB // L):
            sl = pl.ds(j * L, L)
            a = plsc.load_gather(disb, [sch[sl]])
            b = plsc.load_gather(disb, [dch[sl]])
            ob[sl] = a * b * wch[sl]
        pltpu.sync_copy(ob, norm_h.at[pl.ds(ebase + i * CB, CB)])
        return 0
    lax.fori_loop(0, nch, chunk, 0)


def _norm_call(src, dst, w, degp, npad, cnt=None):
    body = functools.partial(_norm_body, npad)
    kfn = pl.kernel(
        (lambda src_h, dst_h, w_h, degp_h, norm_h, *scr:
         body(src_h, dst_h, w_h, degp_h, None, norm_h, *scr))
        if cnt is None else body,
        out_type=jax.ShapeDtypeStruct((NEDGE,), F32),
        mesh=_MESH,
        compiler_params=_SC_PARAMS,
        scratch_types=[
            pltpu.VMEM((npad,), F32),
            pltpu.VMEM((npad,), F32),
            pltpu.VMEM((npad,), F32),
            pltpu.VMEM((CB,), I32),
            pltpu.VMEM((CB,), I32),
            pltpu.VMEM((CB,), F32),
            pltpu.VMEM((CB,), F32),
            pltpu.VMEM((L,), I32),
        ],
    )
    if cnt is None:
        return kfn(src, dst, w, degp)
    return kfn(src, dst, w, degp, cnt)


def _edge_body(npad, dyn, x_h, src_h, dst_h, nrm_h, cnt_h, aggp_h,
               spm, ra, rb, sia, sib, dia, dib, nba, nbb, zb, csm,
               sga, sgb, ssa, ssb):
    c, s, wid = _ids()
    stripe = npad // NS
    row0 = s * stripe
    if dyn:
        nch = _read_nch(cnt_h, csm, wid)
    else:
        nch = NCH
    _zero2d(zb, CB)
    for q in range(stripe // CB):
        pltpu.sync_copy(zb, spm.at[pl.ds(row0 + q * CB, CB)])
    plsc.subcore_barrier()

    ebase = wid * EPW

    def fetch_idx(i, sref):
        pltpu.sync_copy(src_h.at[pl.ds(ebase + i * CB, CB)], sref)

    def process(i, rows, sref, di, nb, gsem, ssem):
        pltpu.sync_copy(dst_h.at[pl.ds(ebase + i * CB, CB)], di)
        pltpu.sync_copy(nrm_h.at[pl.ds(ebase + i * CB, CB)], nb)
        pltpu.make_async_copy(x_h.at[sref], rows, gsem).wait()
        _scale_rows(rows, nb, CB)
        pltpu.async_copy(rows, spm.at[di], ssem, add=True)

    def swait(rows, di, ssem):
        pltpu.make_async_copy(rows, spm.at[di], ssem).wait()

    def refill(i, rows, sref, gsem):
        fetch_idx(i, sref)
        pltpu.async_copy(x_h.at[sref], rows, gsem)

    refill(0, ra, sia, sga)
    refill(1, rb, sib, sgb)

    def pair(ip, _):
        e0 = ip * 2
        process(e0, ra, sia, dia, nba, sga, ssa)
        process(e0 + 1, rb, sib, dib, nbb, sgb, ssb)
        swait(ra, dia, ssa)
        refill(jnp.minimum(e0 + 2, nch - 1), ra, sia, sga)
        swait(rb, dib, ssb)
        refill(jnp.minimum(e0 + 3, nch - 1), rb, sib, sgb)
        return 0

    if not dyn:
        lax.fori_loop(0, (NCH - 1) // 2, pair, 0)
        process(NCH - 1, ra, sia, dia, nba, sga, ssa)
        swait(ra, dia, ssa)
        pltpu.make_async_copy(x_h.at[sib], rb, sgb).wait()
    else:
        lax.fori_loop(0, nch // 2, pair, 0)
        pltpu.make_async_copy(x_h.at[sia], ra, sga).wait()
        pltpu.make_async_copy(x_h.at[sib], rb, sgb).wait()

    plsc.subcore_barrier()
    pltpu.sync_copy(spm.at[pl.ds(row0, stripe)],
                    aggp_h.at[c, pl.ds(row0, stripe)])


def _edge_call(x, src, dst, nrm, npad, cnt=None):
    body = functools.partial(_edge_body, npad, cnt is not None)
    if cnt is None:
        wrapped = (lambda x_h, src_h, dst_h, nrm_h, aggp_h, *scr:
                   body(x_h, src_h, dst_h, nrm_h, None, aggp_h, *scr))
    else:
        wrapped = body
    kfn = pl.kernel(
        wrapped,
        out_type=jax.ShapeDtypeStruct((NC, npad, FD), F32),
        mesh=_MESH,
        compiler_params=_SC_PARAMS,
        scratch_types=[
            pltpu.VMEM_SHARED((npad, FD), F32),
            pltpu.VMEM((CB, FD), F32),
            pltpu.VMEM((CB, FD), F32),
            pltpu.VMEM((CB,), I32),
            pltpu.VMEM((CB,), I32),
            pltpu.VMEM((CB,), I32),
            pltpu.VMEM((CB,), I32),
            pltpu.VMEM((CB,), F32),
            pltpu.VMEM((CB,), F32),
            pltpu.VMEM((CB, FD), F32),
            pltpu.VMEM((L,), I32),
            pltpu.SemaphoreType.DMA,
            pltpu.SemaphoreType.DMA,
            pltpu.SemaphoreType.DMA,
            pltpu.SemaphoreType.DMA,
        ],
    )
    if cnt is None:
        return kfn(x, src, dst, nrm)
    return kfn(x, src, dst, nrm, cnt)


def _sel_body(score_h, h3p_h, enc_h, x2p_h,
              sbuf, kbuf, hb, encb, gb, sxb):
    c, s, wid = _ids()
    nvr = NPAD1 // L
    myvr = 20 * wid

    pltpu.sync_copy(score_h, sbuf)

    def keys(i, _):
        sl = pl.ds(i * L, L)
        v = sbuf[sl]
        b = lax.bitcast_convert_type(v, jnp.uint32)
        kbuf[sl] = jnp.where(v < 0, ~b, b | jnp.uint32(0x80000000))
        return 0
    lax.fori_loop(0, nvr, keys, 0)

    T = jnp.uint32(0)
    for bit in range(31, -1, -1):
        cand = T | jnp.uint32(1 << bit)
        def cnt(i, acc, cand=cand):
            kv = kbuf[pl.ds(i * L, L)]
            return acc + jnp.where(kv >= cand, 1, 0).astype(I32)
        tot = jnp.sum(lax.fori_loop(0, nvr, cnt, jnp.zeros((L,), I32)))
        T = jnp.where(tot >= KSEL, cand, T)

    def cnt3(i, carry):
        mg, pk, pt = carry
        kv = kbuf[pl.ds(i * L, L)]
        gt = jnp.where(kv > T, 1, 0).astype(I32)
        eq = jnp.where(kv == T, 1, 0).astype(I32)
        pre = jnp.where(i < myvr, 1, 0).astype(I32)
        return (mg + gt, pk + gt * pre, pt + eq * pre)
    z = jnp.zeros((L,), I32)
    mg, pk, pt = lax.fori_loop(0, nvr, cnt3, (z, z, z))
    quota = KSEL - jnp.sum(mg)
    tie_ctr = jnp.sum(pt)
    rank_ctr = jnp.sum(pk) + jnp.minimum(tie_ctr, quota)

    for j in range(20):
        sl = pl.ds(wid * 320 + j * L, L)
        kv = kbuf[sl]
        sv = sbuf[sl]
        gt = kv > T
        eqi = jnp.where(kv == T, 1, 0).astype(I32)
        texcl = plsc.cumsum(eqi) - eqi
        tsel = (eqi > 0) & ((tie_ctr + texcl) < quota)
        keep = gt | tsel
        ki = jnp.where(keep, 1, 0).astype(I32)
        rank = rank_ctr + plsc.cumsum(ki) - ki
        osl = pl.ds(j * L, L)
        encb[osl] = jnp.where(keep, rank, -1)
        dump = KSEL + lax.rem(wid * 320 + j * L + lax.iota(I32, L),
                              jnp.int32(128))
        sxb[j // 5, pl.ds((j % 5) * L, L)] = jnp.where(keep, rank, dump)
        e = jnp.exp(2.0 * sv)
        gb[osl] = 1.0 - 2.0 / (e + 1.0)
        tie_ctr = tie_ctr + jnp.sum(eqi)
        rank_ctr = rank_ctr + jnp.sum(ki)

    pltpu.sync_copy(h3p_h.at[pl.ds(wid * 320, 320)], hb)
    _scale_rows(hb, gb, 320)
    for q in range(4):
        pltpu.sync_copy(hb.at[pl.ds(q * CB, CB)], x2p_h.at[sxb.at[q]])
    pltpu.sync_copy(encb, enc_h.at[pl.ds(wid * 320, 320)])


def _sel_call(score, h3p):
    kfn = pl.kernel(
        _sel_body,
        out_type=(jax.ShapeDtypeStruct((NPAD1,), I32),
                  jax.ShapeDtypeStruct((KSEL + 128, FD), F32)),
        mesh=_MESH,
        compiler_params=_SC_PARAMS,
        scratch_types=[
            pltpu.VMEM((NPAD1,), F32),
            pltpu.VMEM((NPAD1,), jnp.uint32),
            pltpu.VMEM((320, FD), F32),
            pltpu.VMEM((320,), I32),
            pltpu.VMEM((320,), F32),
            pltpu.VMEM((4, CB), I32),
        ],
    )
    return kfn(score, h3p)


def _remap_body(src_h, dst_h, w_h, enc_h, s2_h, d2_h, w2_h, deg2p_h, cnt_h,
                enct, degpriv, sch, dch, wch, s2b, d2b, w2b, cb16,
                spm16, rbuf, outb):
    c, s, wid = _ids()
    pltpu.sync_copy(enc_h, enct)
    _fill1d(degpriv, NPAD2, 0.0, F32)
    lanes = lax.iota(I32, L)
    zi = jnp.zeros((L,), I32)
    zf = jnp.zeros((L,), F32)

    def fbody(i, _):
        sl = pl.ds(i * L, L)
        s2b[sl] = zi
        d2b[sl] = KSEL + lax.rem(i * L + lanes, jnp.int32(120))
        w2b[sl] = zf
        return 0
    lax.fori_loop(0, EPW // L, fbody, 0)

    ebase = wid * EPW

    def chunk(i, cur):
        pltpu.sync_copy(src_h.at[pl.ds(ebase + i * CB, CB)], sch)
        pltpu.sync_copy(dst_h.at[pl.ds(ebase + i * CB, CB)], dch)
        pltpu.sync_copy(w_h.at[pl.ds(ebase + i * CB, CB)], wch)
        for j in range(CB // L):
            sl = pl.ds(j * L, L)
            a = plsc.load_gather(enct, [sch[sl]])
            b = plsc.load_gather(enct, [dch[sl]])
            val = (a >= 0) & (b >= 0)
            wv = wch[sl]
            plsc.store_compressed(s2b.at[pl.ds(cur, L)], a, mask=val)
            plsc.store_compressed(d2b.at[pl.ds(cur, L)], b, mask=val)
            plsc.store_compressed(w2b.at[pl.ds(cur, L)], wv, mask=val)
            plsc.addupdate_scatter(degpriv, [b], wv, mask=val)
            cur = cur + plsc.all_reduce_population_count(val)[0]
        return cur
    cnt = lax.fori_loop(0, NCH, chunk, jnp.int32(0))

    nch = 2 * ((cnt + 2 * CB - 1) // (2 * CB))
    nch = jnp.maximum(nch, 2)
    cb16[pl.ds(0, L)] = jnp.zeros((L,), I32) + nch
    pltpu.sync_copy(cb16, cnt_h.at[wid])
    pltpu.sync_copy(s2b, s2_h.at[pl.ds(ebase, EPW)])
    pltpu.sync_copy(d2b, d2_h.at[pl.ds(ebase, EPW)])
    pltpu.sync_copy(w2b, w2_h.at[pl.ds(ebase, EPW)])
    _reduce16_store(degpriv, spm16, rbuf, outb, deg2p_h, c, s, NPAD2)


def _remap_call(src, dst, w, enc):
    stripe, _ = _red_geom(NPAD2)
    kfn = pl.kernel(
        _remap_body,
        out_type=(jax.ShapeDtypeStruct((NEDGE,), I32),
                  jax.ShapeDtypeStruct((NEDGE,), I32),
                  jax.ShapeDtypeStruct((NEDGE,), F32),
                  jax.ShapeDtypeStruct((NC, NPAD2), F32),
                  jax.ShapeDtypeStruct((NW, L), I32)),
        mesh=_MESH,
        compiler_params=_SC_PARAMS,
        scratch_types=[
            pltpu.VMEM((NPAD1,), I32),
            pltpu.VMEM((NPAD2,), F32),
            pltpu.VMEM((CB,), I32),
            pltpu.VMEM((CB,), I32),
            pltpu.VMEM((CB,), F32),
            pltpu.VMEM((EPW,), I32),
            pltpu.VMEM((EPW,), I32),
            pltpu.VMEM((EPW,), F32),
            pltpu.VMEM((L,), I32),
            pltpu.VMEM_SHARED((NS * NPAD2,), F32),
            pltpu.VMEM((NS * stripe,), F32),
            pltpu.VMEM((stripe,), F32),
        ],
    )
    return kfn(src, dst, w, enc)


def _mm_call(a0, a1, W, b, n):
    bm = n // 5
    def body(a0r, a1r, wr, br, outr):
        acc = jnp.dot(a0r[...] + a1r[...], wr[...],
                      preferred_element_type=F32)
        outr[...] = jnp.maximum(acc + br[...], 0.0)
    return pl.pallas_call(
        body,
        grid=(5,),
        in_specs=[
            pl.BlockSpec((bm, FD), lambda i: (i, 0)),
            pl.BlockSpec((bm, FD), lambda i: (i, 0)),
            pl.BlockSpec((FD, FD), lambda i: (0, 0)),
            pl.BlockSpec((1, FD), lambda i: (0, 0)),
        ],
        out_specs=pl.BlockSpec((bm, FD), lambda i: (i, 0)),
        out_shape=jax.ShapeDtypeStruct((n, FD), F32),
    )(a0, a1, W, b.reshape(1, FD))


def _mm_score_call(a0, a1, W, b, p, n):
    bm = n // 5
    def body(a0r, a1r, wr, br, pr, outr, scr):
        acc = jnp.dot(a0r[...] + a1r[...], wr[...],
                      preferred_element_type=F32)
        h = jnp.maximum(acc + br[...], 0.0)
        outr[...] = h
        pv = pr[...]
        pn = pv / (jnp.sqrt(jnp.sum(pv * pv)) + 1e-12)
        scr[...] = jnp.dot(h, pn, preferred_element_type=F32)
    return pl.pallas_call(
        body,
        grid=(5,),
        in_specs=[
            pl.BlockSpec((bm, FD), lambda i: (i, 0)),
            pl.BlockSpec((bm, FD), lambda i: (i, 0)),
            pl.BlockSpec((FD, FD), lambda i: (0, 0)),
            pl.BlockSpec((1, FD), lambda i: (0, 0)),
            pl.BlockSpec((FD, 1), lambda i: (0, 0)),
        ],
        out_specs=[
            pl.BlockSpec((bm, FD), lambda i: (i, 0)),
            pl.BlockSpec((bm, 1), lambda i: (i, 0)),
        ],
        out_shape=[
            jax.ShapeDtypeStruct((n, FD), F32),
            jax.ShapeDtypeStruct((n, 1), F32),
        ],
    )(a0, a1, W, b.reshape(1, FD), p.reshape(FD, 1))


def _head_call(h, Wl1, bl1, Wl2, bl2):
    def body(hr, w1r, b1r, w2r, b2r, outr):
        g = jnp.sum(hr[...], axis=0, keepdims=True) * (1.0 / KSEL)
        z = jnp.maximum(
            jnp.dot(g, w1r[...], preferred_element_type=F32) + b1r[...], 0.0)
        o = jnp.dot(z, w2r[...], preferred_element_type=F32) + b2r[...]
        outr[...] = jax.nn.sigmoid(o)
    return pl.pallas_call(
        body,
        out_shape=jax.ShapeDtypeStruct((1, NOUT), F32),
    )(h, Wl1, bl1.reshape(1, -1), Wl2, bl2.reshape(1, -1))


def kernel(x, edge_index, adj, W1, b1, W2, b2, W3, b3, V1, v1, V2, v2, V3, v3,
           U1, u1, U2, u2, U3, u3, p, Wl1, bl1, Wl2, bl2):
    src = edge_index[0]
    dst = edge_index[1]

    degp1 = _deg_call(dst, adj, NPAD1)
    norm1 = _norm_call(src, dst, adj, degp1, NPAD1)

    h = x
    aggp = _edge_call(h, src, dst, norm1, NPAD1)
    h = _mm_call(aggp[0, :NNODE], aggp[1, :NNODE], W1, b1, NNODE)
    aggp = _edge_call(h, src, dst, norm1, NPAD1)
    h = _mm_call(aggp[0, :NNODE], aggp[1, :NNODE], W2, b2, NNODE)
    aggp = _edge_call(h, src, dst, norm1, NPAD1)
    h3, score = _mm_score_call(aggp[0, :NNODE], aggp[1, :NNODE], W3, b3, p,
                               NNODE)

    h3p = jnp.pad(h3, ((0, NPAD1 - NNODE), (0, 0)))
    scorep = jnp.pad(score.reshape(NNODE), (0, NPAD1 - NNODE),
                     constant_values=-3.4e38)
    enc, x2p = _sel_call(scorep, h3p)
    x2 = x2p[:KSEL]

    s2, d2, w2, deg2p, ecnt = _remap_call(src, dst, adj, enc)
    norm2 = _norm_call(s2, d2, w2, deg2p, NPAD2, ecnt)

    h = x2
    for (Wp, bp) in [(V1, v1), (V2, v2), (V3, v3), (U1, u1), (U2, u2),
                     (U3, u3)]:
        aggp = _edge_call(h, s2, d2, norm2, NPAD2, ecnt)
        h = _mm_call(aggp[0, :KSEL], aggp[1, :KSEL], Wp, bp, KSEL)

    out = _head_call(h, Wl1, bl1, Wl2, bl2)
    return out.reshape(NOUT)


# speedup vs baseline: 1.1766x; 1.1766x over previous
import functools

import jax
import jax.numpy as jnp
from jax import lax
from jax.experimental import pallas as pl
from jax.experimental.pallas import tpu as pltpu
from jax.experimental.pallas import tpu_sc as plsc

F32 = jnp.float32
I32 = jnp.int32

NC, NS, L = 2, 16, 16
NW = NC * NS
NNODE = 10000
NEDGE = 320000
FD = 128
KSEL = 5000
NOUT = 1000
NPAD1 = 10240
NPAD2 = 5120
EPW = NEDGE // NW
CB = 80
NCH = EPW // CB

_MESH = plsc.VectorSubcoreMesh(
    core_axis_name="c", subcore_axis_name="s", num_cores=NC, num_subcores=NS)
_SC_PARAMS = pltpu.CompilerParams(needs_layout_passes=False)


def _ids():
    c = lax.axis_index("c")
    s = lax.axis_index("s")
    return c, s, c * NS + s


def _fill1d(ref, n, val, dtype):
    v = jnp.full((L,), val, dtype)
    def body(i, _):
        ref[pl.ds(i * L, L)] = v
        return 0
    lax.fori_loop(0, n // L, body, 0)


def _zero2d(ref, rows):
    z = jnp.zeros((L,), F32)
    def body(i, _):
        for j in range(FD // L):
            ref[i, pl.ds(j * L, L)] = z
        return 0
    lax.fori_loop(0, rows, body, 0)


def _scale_rows(rows_ref, sc_ref, nrows):
    def body(r, _):
        gvec = sc_ref[pl.ds(r * L, L)]
        for t in range(L):
            i = r * L + t
            g = gvec[t]
            for j in range(FD // L):
                sl = (i, pl.ds(j * L, L))
                rows_ref[sl] = rows_ref[sl] * g
        return 0
    lax.fori_loop(0, nrows // L, body, 0)


def _red_geom(npad):
    stripe = npad // NS
    if stripe % 128 != 0:
        stripe = 512
    return stripe, npad // stripe


def _reduce16_store(degpriv, spm16, rbuf, outb, out_hbm, c, s, npad):
    stripe, nact = _red_geom(npad)
    pltpu.sync_copy(degpriv, spm16.at[pl.ds(s * npad, npad)])
    plsc.subcore_barrier()

    @pl.when(s < nact)
    def _():
        for t in range(NS):
            pltpu.sync_copy(spm16.at[pl.ds(t * npad + stripe * s, stripe)],
                            rbuf.at[pl.ds(t * stripe, stripe)])
        def body(j, _):
            acc = rbuf[pl.ds(j * L, L)]
            for t in range(1, NS):
                acc = acc + rbuf[pl.ds(t * stripe + j * L, L)]
            outb[pl.ds(j * L, L)] = acc
            return 0
        lax.fori_loop(0, stripe // L, body, 0)
        pltpu.sync_copy(outb, out_hbm.at[c, pl.ds(stripe * s, stripe)])


def _read_nch(cnt_h, csm, wid):
    pltpu.sync_copy(cnt_h.at[wid], csm)
    return csm[pl.ds(0, L)][0]


def _deg_body(npad, dst_h, w_h, degp_h, dch, wch, degpriv, spm16, rbuf, outb):
    c, s, wid = _ids()
    _fill1d(degpriv, npad, 0.0, F32)
    ebase = wid * EPW
    def chunk(i, _):
        pltpu.sync_copy(dst_h.at[pl.ds(ebase + i * CB, CB)], dch)
        pltpu.sync_copy(w_h.at[pl.ds(ebase + i * CB, CB)], wch)
        for j in range(CB // L):
            dv = dch[pl.ds(j * L, L)]
            wv = wch[pl.ds(j * L, L)]
            plsc.addupdate_scatter(degpriv, [dv], wv)
        return 0
    lax.fori_loop(0, NCH, chunk, 0)
    _reduce16_store(degpriv, spm16, rbuf, outb, degp_h, c, s, npad)


def _deg_call(dst, w, npad):
    stripe, _ = _red_geom(npad)
    body = functools.partial(_deg_body, npad)
    kfn = pl.kernel(
        body,
        out_type=jax.ShapeDtypeStruct((NC, npad), F32),
        mesh=_MESH,
        compiler_params=_SC_PARAMS,
        scratch_types=[
            pltpu.VMEM((CB,), I32),
            pltpu.VMEM((CB,), F32),
            pltpu.VMEM((npad,), F32),
            pltpu.VMEM_SHARED((NS * npad,), F32),
            pltpu.VMEM((NS * stripe,), F32),
            pltpu.VMEM((stripe,), F32),
        ],
    )
    return kfn(dst, w)


def _rsqrt_guard(d):
    dm = jnp.maximum(d, 1e-12)
    bits = lax.bitcast_convert_type(dm, I32)
    y = lax.bitcast_convert_type(jnp.int32(0x5F3759DF) - (bits >> 1), F32)
    for _ in range(3):
        y = y * (1.5 - 0.5 * dm * y * y)
    return jnp.where(d > 0, y, 0.0)


def _norm_body(npad, src_h, dst_h, w_h, degp_h, cnt_h, norm_h,
               da, db, disb, sch, dch, wch, ob, csm):
    c, s, wid = _ids()
    if cnt_h is None:
        nch = NCH
    else:
        nch = _read_nch(cnt_h, csm, wid)
    pltpu.sync_copy(degp_h.at[0], da)
    pltpu.sync_copy(degp_h.at[1], db)
    def dbody(i, _):
        sl = pl.ds(i * L, L)
        disb[sl] = _rsqrt_guard(da[sl] + db[sl])
        return 0
    lax.fori_loop(0, npad // L, dbody, 0)
    ebase = wid * EPW
    def chunk(i, _):
        pltpu.sync_copy(src_h.at[pl.ds(ebase + i * CB, CB)], sch)
        pltpu.sync_copy(dst_h.at[pl.ds(ebase + i * CB, CB)], dch)
        pltpu.sync_copy(w_h.at[pl.ds(ebase + i * CB, CB)], wch)
        for j in range(CB // L):
            sl = pl.ds(j * L, L)
            a = plsc.load_gather(disb, [sch[sl]])
            b = plsc.load_gather(disb, [dch[sl]])
            ob[sl] = a * b * wch[sl]
        pltpu.sync_copy(ob, norm_h.at[pl.ds(ebase + i * CB, CB)])
        return 0
    lax.fori_loop(0, nch, chunk, 0)


def _norm_call(src, dst, w, degp, npad, cnt=None):
    body = functools.partial(_norm_body, npad)
    kfn = pl.kernel(
        (lambda src_h, dst_h, w_h, degp_h, norm_h, *scr:
         body(src_h, dst_h, w_h, degp_h, None, norm_h, *scr))
        if cnt is None else body,
        out_type=jax.ShapeDtypeStruct((NEDGE,), F32),
        mesh=_MESH,
        compiler_params=_SC_PARAMS,
        scratch_types=[
            pltpu.VMEM((npad,), F32),
            pltpu.VMEM((npad,), F32),
            pltpu.VMEM((npad,), F32),
            pltpu.VMEM((CB,), I32),
            pltpu.VMEM((CB,), I32),
            pltpu.VMEM((CB,), F32),
            pltpu.VMEM((CB,), F32),
            pltpu.VMEM((L,), I32),
        ],
    )
    if cnt is None:
        return kfn(src, dst, w, degp)
    return kfn(src, dst, w, degp, cnt)


def _edge_body(npad, dyn, x_h, src_h, dst_h, nrm_h, cnt_h, aggp_h,
               spm, ra, rb, sia, sib, di, nb, zb, csm, sga, sgb):
    c, s, wid = _ids()
    stripe = npad // NS
    row0 = s * stripe
    if dyn:
        nch = _read_nch(cnt_h, csm, wid)
    _zero2d(zb, CB)
    for q in range(stripe // CB):
        pltpu.sync_copy(zb, spm.at[pl.ds(row0 + q * CB, CB)])
    plsc.subcore_barrier()

    ebase = wid * EPW

    def fetch_idx(i, sref):
        pltpu.sync_copy(src_h.at[pl.ds(ebase + i * CB, CB)], sref)

    def process(i, rows, sref, sem):
        pltpu.sync_copy(dst_h.at[pl.ds(ebase + i * CB, CB)], di)
        pltpu.sync_copy(nrm_h.at[pl.ds(ebase + i * CB, CB)], nb)
        pltpu.make_async_copy(x_h.at[sref], rows, sem).wait()
        _scale_rows(rows, nb, CB)
        pltpu.sync_copy(rows, spm.at[di], add=True)

    fetch_idx(0, sia)
    pltpu.async_copy(x_h.at[sia], ra, sga)

    def pair(ip, _):
        e0 = ip * 2
        fetch_idx(e0 + 1, sib)
        pltpu.async_copy(x_h.at[sib], rb, sgb)
        process(e0, ra, sia, sga)
        fetch_idx(e0 + 2, sia)
        pltpu.async_copy(x_h.at[sia], ra, sga)
        process(e0 + 1, rb, sib, sgb)
        return 0

    if not dyn:
        lax.fori_loop(0, (NCH - 1) // 2, pair, 0)
        process(NCH - 1, ra, sia, sga)
    else:
        def pairc(ip, _):
            e0 = ip * 2
            fetch_idx(e0 + 1, sib)
            pltpu.async_copy(x_h.at[sib], rb, sgb)
            process(e0, ra, sia, sga)
            nxt = jnp.minimum(e0 + 2, nch - 2)
            fetch_idx(nxt, sia)
            pltpu.async_copy(x_h.at[sia], ra, sga)
            process(e0 + 1, rb, sib, sgb)
            return 0
        lax.fori_loop(0, nch // 2, pairc, 0)
        pltpu.make_async_copy(x_h.at[sia], ra, sga).wait()

    plsc.subcore_barrier()
    pltpu.sync_copy(spm.at[pl.ds(row0, stripe)],
                    aggp_h.at[c, pl.ds(row0, stripe)])


def _edge_call(x, src, dst, nrm, npad, cnt=None):
    body = functools.partial(_edge_body, npad, cnt is not None)
    if cnt is None:
        wrapped = (lambda x_h, src_h, dst_h, nrm_h, aggp_h, *scr:
                   body(x_h, src_h, dst_h, nrm_h, None, aggp_h, *scr))
    else:
        wrapped = body
    kfn = pl.kernel(
        wrapped,
        out_type=jax.ShapeDtypeStruct((NC, npad, FD), F32),
        mesh=_MESH,
        compiler_params=_SC_PARAMS,
        scratch_types=[
            pltpu.VMEM_SHARED((npad, FD), F32),
            pltpu.VMEM((CB, FD), F32),
            pltpu.VMEM((CB, FD), F32),
            pltpu.VMEM((CB,), I32),
            pltpu.VMEM((CB,), I32),
            pltpu.VMEM((CB,), I32),
            pltpu.VMEM((CB,), F32),
            pltpu.VMEM((CB, FD), F32),
            pltpu.VMEM((L,), I32),
            pltpu.SemaphoreType.DMA,
            pltpu.SemaphoreType.DMA,
        ],
    )
    if cnt is None:
        return kfn(x, src, dst, nrm)
    return kfn(x, src, dst, nrm, cnt)


def _sel_body(score_h, h3p_h, enc_h, x2p_h,
              sbuf, kbuf, hb, encb, gb, sxb):
    c, s, wid = _ids()
    nvr = NPAD1 // L
    myvr = 20 * wid

    pltpu.sync_copy(score_h, sbuf)

    def keys(i, _):
        sl = pl.ds(i * L, L)
        v = sbuf[sl]
        b = lax.bitcast_convert_type(v, jnp.uint32)
        kbuf[sl] = jnp.where(v < 0, ~b, b | jnp.uint32(0x80000000))
        return 0
    lax.fori_loop(0, nvr, keys, 0)

    T = jnp.uint32(0)
    for bit in range(31, -1, -1):
        cand = T | jnp.uint32(1 << bit)
        def cnt(i, acc, cand=cand):
            kv = kbuf[pl.ds(i * L, L)]
            return acc + jnp.where(kv >= cand, 1, 0).astype(I32)
        tot = jnp.sum(lax.fori_loop(0, nvr, cnt, jnp.zeros((L,), I32)))
        T = jnp.where(tot >= KSEL, cand, T)

    def cnt3(i, carry):
        mg, pk, pt = carry
        kv = kbuf[pl.ds(i * L, L)]
        gt = jnp.where(kv > T, 1, 0).astype(I32)
        eq = jnp.where(kv == T, 1, 0).astype(I32)
        pre = jnp.where(i < myvr, 1, 0).astype(I32)
        return (mg + gt, pk + gt * pre, pt + eq * pre)
    z = jnp.zeros((L,), I32)
    mg, pk, pt = lax.fori_loop(0, nvr, cnt3, (z, z, z))
    quota = KSEL - jnp.sum(mg)
    tie_ctr = jnp.sum(pt)
    rank_ctr = jnp.sum(pk) + jnp.minimum(tie_ctr, quota)

    for j in range(20):
        sl = pl.ds(wid * 320 + j * L, L)
        kv = kbuf[sl]
        sv = sbuf[sl]
        gt = kv > T
        eqi = jnp.where(kv == T, 1, 0).astype(I32)
        texcl = plsc.cumsum(eqi) - eqi
        tsel = (eqi > 0) & ((tie_ctr + texcl) < quota)
        keep = gt | tsel
        ki = jnp.where(keep, 1, 0).astype(I32)
        rank = rank_ctr + plsc.cumsum(ki) - ki
        osl = pl.ds(j * L, L)
        encb[osl] = jnp.where(keep, rank, -1)
        dump = KSEL + lax.rem(wid * 320 + j * L + lax.iota(I32, L),
                              jnp.int32(128))
        sxb[j // 5, pl.ds((j % 5) * L, L)] = jnp.where(keep, rank, dump)
        e = jnp.exp(2.0 * sv)
        gb[osl] = 1.0 - 2.0 / (e + 1.0)
        tie_ctr = tie_ctr + jnp.sum(eqi)
        rank_ctr = rank_ctr + jnp.sum(ki)

    pltpu.sync_copy(h3p_h.at[pl.ds(wid * 320, 320)], hb)
    _scale_rows(hb, gb, 320)
    for q in range(4):
        pltpu.sync_copy(hb.at[pl.ds(q * CB, CB)], x2p_h.at[sxb.at[q]])
    pltpu.sync_copy(encb, enc_h.at[pl.ds(wid * 320, 320)])


def _sel_call(score, h3p):
    kfn = pl.kernel(
        _sel_body,
        out_type=(jax.ShapeDtypeStruct((NPAD1,), I32),
                  jax.ShapeDtypeStruct((KSEL + 128, FD), F32)),
        mesh=_MESH,
        compiler_params=_SC_PARAMS,
        scratch_types=[
            pltpu.VMEM((NPAD1,), F32),
            pltpu.VMEM((NPAD1,), jnp.uint32),
            pltpu.VMEM((320, FD), F32),
            pltpu.VMEM((320,), I32),
            pltpu.VMEM((320,), F32),
            pltpu.VMEM((4, CB), I32),
        ],
    )
    return kfn(score, h3p)


def _remap_body(src_h, dst_h, w_h, enc_h, s2_h, d2_h, w2_h, deg2p_h, cnt_h,
                enct, degpriv, sch, dch, wch, s2b, d2b, w2b, cb16,
                spm16, rbuf, outb):
    c, s, wid = _ids()
    pltpu.sync_copy(enc_h, enct)
    _fill1d(degpriv, NPAD2, 0.0, F32)
    lanes = lax.iota(I32, L)
    zi = jnp.zeros((L,), I32)
    zf = jnp.zeros((L,), F32)

    def fbody(i, _):
        sl = pl.ds(i * L, L)
        s2b[sl] = zi
        d2b[sl] = KSEL + lax.rem(i * L + lanes, jnp.int32(120))
        w2b[sl] = zf
        return 0
    lax.fori_loop(0, EPW // L, fbody, 0)

    ebase = wid * EPW

    def chunk(i, cur):
        pltpu.sync_copy(src_h.at[pl.ds(ebase + i * CB, CB)], sch)
        pltpu.sync_copy(dst_h.at[pl.ds(ebase + i * CB, CB)], dch)
        pltpu.sync_copy(w_h.at[pl.ds(ebase + i * CB, CB)], wch)
        for j in range(CB // L):
            sl = pl.ds(j * L, L)
            a = plsc.load_gather(enct, [sch[sl]])
            b = plsc.load_gather(enct, [dch[sl]])
            val = (a >= 0) & (b >= 0)
            wv = wch[sl]
            plsc.store_compressed(s2b.at[pl.ds(cur, L)], a, mask=val)
            plsc.store_compressed(d2b.at[pl.ds(cur, L)], b, mask=val)
            plsc.store_compressed(w2b.at[pl.ds(cur, L)], wv, mask=val)
            plsc.addupdate_scatter(degpriv, [b], wv, mask=val)
            cur = cur + plsc.all_reduce_population_count(val)[0]
        return cur
    cnt = lax.fori_loop(0, NCH, chunk, jnp.int32(0))

    nch = 2 * ((cnt + 2 * CB - 1) // (2 * CB))
    nch = jnp.maximum(nch, 2)
    cb16[pl.ds(0, L)] = jnp.zeros((L,), I32) + nch
    pltpu.sync_copy(cb16, cnt_h.at[wid])
    pltpu.sync_copy(s2b, s2_h.at[pl.ds(ebase, EPW)])
    pltpu.sync_copy(d2b, d2_h.at[pl.ds(ebase, EPW)])
    pltpu.sync_copy(w2b, w2_h.at[pl.ds(ebase, EPW)])
    _reduce16_store(degpriv, spm16, rbuf, outb, deg2p_h, c, s, NPAD2)


def _remap_call(src, dst, w, enc):
    stripe, _ = _red_geom(NPAD2)
    kfn = pl.kernel(
        _remap_body,
        out_type=(jax.ShapeDtypeStruct((NEDGE,), I32),
                  jax.ShapeDtypeStruct((NEDGE,), I32),
                  jax.ShapeDtypeStruct((NEDGE,), F32),
                  jax.ShapeDtypeStruct((NC, NPAD2), F32),
                  jax.ShapeDtypeStruct((NW, L), I32)),
        mesh=_MESH,
        compiler_params=_SC_PARAMS,
        scratch_types=[
            pltpu.VMEM((NPAD1,), I32),
            pltpu.VMEM((NPAD2,), F32),
            pltpu.VMEM((CB,), I32),
            pltpu.VMEM((CB,), I32),
            pltpu.VMEM((CB,), F32),
            pltpu.VMEM((EPW,), I32),
            pltpu.VMEM((EPW,), I32),
            pltpu.VMEM((EPW,), F32),
            pltpu.VMEM((L,), I32),
            pltpu.VMEM_SHARED((NS * NPAD2,), F32),
            pltpu.VMEM((NS * stripe,), F32),
            pltpu.VMEM((stripe,), F32),
        ],
    )
    return kfn(src, dst, w, enc)


def _mm_call(a0, a1, W, b, n):
    bm = n // 5
    def body(a0r, a1r, wr, br, outr):
        acc = jnp.dot(a0r[...] + a1r[...], wr[...],
                      preferred_element_type=F32)
        outr[...] = jnp.maximum(acc + br[...], 0.0)
    return pl.pallas_call(
        body,
        grid=(5,),
        in_specs=[
            pl.BlockSpec((bm, FD), lambda i: (i, 0)),
            pl.BlockSpec((bm, FD), lambda i: (i, 0)),
            pl.BlockSpec((FD, FD), lambda i: (0, 0)),
            pl.BlockSpec((1, FD), lambda i: (0, 0)),
        ],
        out_specs=pl.BlockSpec((bm, FD), lambda i: (i, 0)),
        out_shape=jax.ShapeDtypeStruct((n, FD), F32),
    )(a0, a1, W, b.reshape(1, FD))


def _mm_score_call(a0, a1, W, b, p, n):
    bm = n // 5
    def body(a0r, a1r, wr, br, pr, outr, scr):
        acc = jnp.dot(a0r[...] + a1r[...], wr[...],
                      preferred_element_type=F32)
        h = jnp.maximum(acc + br[...], 0.0)
        outr[...] = h
        pv = pr[...]
        pn = pv / (jnp.sqrt(jnp.sum(pv * pv)) + 1e-12)
        scr[...] = jnp.dot(h, pn, preferred_element_type=F32)
    return pl.pallas_call(
        body,
        grid=(5,),
        in_specs=[
            pl.BlockSpec((bm, FD), lambda i: (i, 0)),
            pl.BlockSpec((bm, FD), lambda i: (i, 0)),
            pl.BlockSpec((FD, FD), lambda i: (0, 0)),
            pl.BlockSpec((1, FD), lambda i: (0, 0)),
            pl.BlockSpec((FD, 1), lambda i: (0, 0)),
        ],
        out_specs=[
            pl.BlockSpec((bm, FD), lambda i: (i, 0)),
            pl.BlockSpec((bm, 1), lambda i: (i, 0)),
        ],
        out_shape=[
            jax.ShapeDtypeStruct((n, FD), F32),
            jax.ShapeDtypeStruct((n, 1), F32),
        ],
    )(a0, a1, W, b.reshape(1, FD), p.reshape(FD, 1))


def _head_call(h, Wl1, bl1, Wl2, bl2):
    def body(hr, w1r, b1r, w2r, b2r, outr):
        g = jnp.sum(hr[...], axis=0, keepdims=True) * (1.0 / KSEL)
        z = jnp.maximum(
            jnp.dot(g, w1r[...], preferred_element_type=F32) + b1r[...], 0.0)
        o = jnp.dot(z, w2r[...], preferred_element_type=F32) + b2r[...]
        outr[...] = jax.nn.sigmoid(o)
    return pl.pallas_call(
        body,
        out_shape=jax.ShapeDtypeStruct((1, NOUT), F32),
    )(h, Wl1, bl1.reshape(1, -1), Wl2, bl2.reshape(1, -1))


def kernel(x, edge_index, adj, W1, b1, W2, b2, W3, b3, V1, v1, V2, v2, V3, v3,
           U1, u1, U2, u2, U3, u3, p, Wl1, bl1, Wl2, bl2):
    src = edge_index[0]
    dst = edge_index[1]

    degp1 = _deg_call(dst, adj, NPAD1)
    norm1 = _norm_call(src, dst, adj, degp1, NPAD1)

    h = x
    aggp = _edge_call(h, src, dst, norm1, NPAD1)
    h = _mm_call(aggp[0, :NNODE], aggp[1, :NNODE], W1, b1, NNODE)
    aggp = _edge_call(h, src, dst, norm1, NPAD1)
    h = _mm_call(aggp[0, :NNODE], aggp[1, :NNODE], W2, b2, NNODE)
    aggp = _edge_call(h, src, dst, norm1, NPAD1)
    h3, score = _mm_score_call(aggp[0, :NNODE], aggp[1, :NNODE], W3, b3, p,
                               NNODE)

    h3p = jnp.pad(h3, ((0, NPAD1 - NNODE), (0, 0)))
    scorep = jnp.pad(score.reshape(NNODE), (0, NPAD1 - NNODE),
                     constant_values=-3.4e38)
    enc, x2p = _sel_call(scorep, h3p)
    x2 = x2p[:KSEL]

    s2, d2, w2, deg2p, ecnt = _remap_call(src, dst, adj, enc)
    norm2 = _norm_call(s2, d2, w2, deg2p, NPAD2, ecnt)

    h = x2
    for (Wp, bp) in [(V1, v1), (V2, v2), (V3, v3), (U1, u1), (U2, u2),
                     (U3, u3)]:
        aggp = _edge_call(h, s2, d2, norm2, NPAD2, ecnt)
        h = _mm_call(aggp[0, :KSEL], aggp[1, :KSEL], Wp, bp, KSEL)

    out = _head_call(h, Wl1, bl1, Wl2, bl2)
    return out.reshape(NOUT)
